# Initial kernel scaffold; baseline (speedup 1.0000x reference)
import jax
import jax.numpy as jnp
from jax import lax
from jax.experimental import pallas as pl
from jax.experimental.pallas import tpu as pltpu
from jax.experimental.pallas import tpu_sc as plsc

N = 10000
E = 320000
D = 128
G = 64

NC = 2
NS = 16
NW = NC * NS

CHUNK = 128
SUPER = 4
EDGES_PER_SUPER = CHUNK * SUPER * NW
E_PAD = ((E + EDGES_PER_SUPER - 1) // EDGES_PER_SUPER) * EDGES_PER_SUPER
ROWS_TOTAL = E_PAD // CHUNK
ROWS_PER_W = ROWS_TOTAL // NW
SUPERS_PER_W = ROWS_PER_W // SUPER
N_ACC = N + 16
ROWS_PER_S = N // NS
ZROWS_PER_S = N_ACC // NS


def _agg_body(h_hbm, src_hbm, dst_hbm, out_hbm, src_v, dst_v, rows_v, acc_sh, sem):
    c = lax.axis_index("c")
    s = lax.axis_index("s")
    w = c * NS + s

    zbuf_rows = rows_v.shape[0]

    def zero_body(i, _):
        r = i // (D // 16)
        cc = i % (D // 16)
        rows_v[r, pl.ds(cc * 16, 16)] = jnp.zeros((16,), jnp.float32)
        return 0

    lax.fori_loop(0, zbuf_rows * (D // 16), zero_body, 0)
    z0 = s * ZROWS_PER_S
    pltpu.sync_copy(rows_v.at[pl.ds(0, 512)], acc_sh.at[pl.ds(z0, 512)])
    pltpu.sync_copy(rows_v.at[pl.ds(0, ZROWS_PER_S - 512)],
                    acc_sh.at[pl.ds(z0 + 512, ZROWS_PER_S - 512)])
    plsc.subcore_barrier()

    w_row0 = w * ROWS_PER_W

    def super_body(g, _):
        row0 = w_row0 + g * SUPER
        pltpu.sync_copy(src_hbm.at[pl.ds(row0, SUPER)], src_v)
        pltpu.sync_copy(dst_hbm.at[pl.ds(row0, SUPER)], dst_v)
        cps = [
            pltpu.async_copy(h_hbm.at[src_v.at[j]],
                             rows_v.at[pl.ds(j * CHUNK, CHUNK)], sem)
            for j in range(SUPER)
        ]
        for cp in cps:
            cp.wait()
        for j in range(SUPER):
            pltpu.sync_copy(rows_v.at[pl.ds(j * CHUNK, CHUNK)],
                            acc_sh.at[dst_v.at[j]], add=True)
        return 0

    lax.fori_loop(0, SUPERS_PER_W, super_body, 0)
    plsc.subcore_barrier()

    r0 = s * ROWS_PER_S
    pltpu.sync_copy(acc_sh.at[pl.ds(r0, ROWS_PER_S)],
                    out_hbm.at[c, pl.ds(r0, ROWS_PER_S)])


_agg = pl.kernel(
    _agg_body,
    out_type=jax.ShapeDtypeStruct((NC, N, D), jnp.float32),
    mesh=plsc.VectorSubcoreMesh(core_axis_name="c", subcore_axis_name="s"),
    scratch_types=[
        pltpu.VMEM((SUPER, CHUNK), jnp.int32),
        pltpu.VMEM((SUPER, CHUNK), jnp.int32),
        pltpu.VMEM((SUPER * CHUNK, D), jnp.float32),
        pltpu.VMEM_SHARED((N_ACC, D), jnp.float32),
        pltpu.SemaphoreType.DMA,
    ],
)


_BN = 1000


def _combine_body(h_ref, p0_ref, p1_ref, w_ref, b_ref, o_ref):
    z = h_ref[...] + p0_ref[...] + p1_ref[...]
    y = jnp.dot(z, w_ref[...], preferred_element_type=jnp.float32) + b_ref[...]
    o_ref[...] = jnp.where(y > 0, y, jnp.expm1(y))


def _combine(h, p0, p1, wT, b):
    nb = N // _BN
    return pl.pallas_call(
        _combine_body,
        grid=(nb,),
        in_specs=[
            pl.BlockSpec((_BN, D), lambda i: (i, 0)),
            pl.BlockSpec((_BN, D), lambda i: (i, 0)),
            pl.BlockSpec((_BN, D), lambda i: (i, 0)),
            pl.BlockSpec((D, D), lambda i: (0, 0)),
            pl.BlockSpec((1, D), lambda i: (0, 0)),
        ],
        out_specs=pl.BlockSpec((_BN, D), lambda i: (i, 0)),
        out_shape=jax.ShapeDtypeStruct((N, D), jnp.float32),
    )(h, p0, p1, wT, b)


OUT = 2


def _pool_body(h_ref, b_ref, wo1_ref, bo1_ref, wo2_ref, bo2_ref, o_ref, acc_ref):
    i = pl.program_id(0)

    @pl.when(i == 0)
    def _():
        acc_ref[...] = jnp.zeros_like(acc_ref)

    seg = b_ref[0, 0, :]
    onehot = (lax.broadcasted_iota(jnp.int32, (G, _BN), 0) == seg[None, :])
    acc_ref[...] += jnp.dot(onehot.astype(jnp.float32), h_ref[...],
                            preferred_element_type=jnp.float32)

    @pl.when(i == N // _BN - 1)
    def _():
        t = jnp.dot(acc_ref[...], wo1_ref[...],
                    preferred_element_type=jnp.float32) + bo1_ref[...]
        o_ref[...] = jnp.dot(t, wo2_ref[...],
                             preferred_element_type=jnp.float32) + bo2_ref[...]


def _pool(h, batch3d, wo1T, bo1, wo2T, bo2):
    nb = N // _BN
    return pl.pallas_call(
        _pool_body,
        grid=(nb,),
        in_specs=[
            pl.BlockSpec((_BN, D), lambda i: (i, 0)),
            pl.BlockSpec((1, 1, _BN), lambda i: (i, 0, 0)),
            pl.BlockSpec((D, D), lambda i: (0, 0)),
            pl.BlockSpec((1, D), lambda i: (0, 0)),
            pl.BlockSpec((D, OUT), lambda i: (0, 0)),
            pl.BlockSpec((1, OUT), lambda i: (0, 0)),
        ],
        out_specs=pl.BlockSpec((G, OUT), lambda i: (0, 0)),
        out_shape=jax.ShapeDtypeStruct((G, OUT), jnp.float32),
        scratch_shapes=[pltpu.VMEM((G, D), jnp.float32)],
    )(h, batch3d, wo1T, bo1, wo2T, bo2)


def kernel(x, edge_index, batch, W1, b1, W2, b2, W3, b3, Wo1, bo1, Wo2, bo2):
    src = edge_index[0]
    dst = edge_index[1]
    pad = E_PAD - E
    src2d = jnp.concatenate([src, jnp.zeros((pad,), jnp.int32)]).reshape(ROWS_TOTAL, CHUNK)
    dst2d = jnp.concatenate([dst, jnp.full((pad,), N, jnp.int32)]).reshape(ROWS_TOTAL, CHUNK)
    batch3d = batch.reshape(N // _BN, 1, _BN)

    h = x
    for W, b in ((W1, b1), (W2, b2), (W3, b3)):
        part = _agg(h, src2d, dst2d)
        h = _combine(h, part[0], part[1], W.T, b.reshape(1, D))
    return _pool(h, batch3d, Wo1.T, bo1.reshape(1, D), Wo2.T, bo2.reshape(1, OUT))


# speedup vs baseline: 2.7166x; 2.7166x over previous
import jax
import jax.numpy as jnp
from jax import lax
from jax.experimental import pallas as pl
from jax.experimental.pallas import tpu as pltpu
from jax.experimental.pallas import tpu_sc as plsc

N = 10000
E = 320000
D = 128
G = 64

NC = 2
NS = 16
NW = NC * NS

CHUNK = 128
SUPER = 8
GROUP = 2
EDGES_PER_SUPER = CHUNK * SUPER * NW
E_PAD = ((E + EDGES_PER_SUPER - 1) // EDGES_PER_SUPER) * EDGES_PER_SUPER
ROWS_TOTAL = E_PAD // CHUNK
ROWS_PER_W = ROWS_TOTAL // NW
SUPERS_PER_W = ROWS_PER_W // SUPER
N_ACC = 10240
ZROWS_PER_S = N_ACC // NS
WB_ROWS = 624
WB_LAST = N - (NS - 1) * WB_ROWS
RB = GROUP * CHUNK


def _agg_body(h_hbm, src_hbm, dst_hbm, out_hbm, src_v, dst_v, rows_v, acc_sh, sem):
    c = lax.axis_index("c")
    s = lax.axis_index("s")
    w = c * NS + s

    def zero_body(i, _):
        r = i // (D // 16)
        cc = i % (D // 16)
        rows_v[r, pl.ds(cc * 16, 16)] = jnp.zeros((16,), jnp.float32)
        return 0

    lax.fori_loop(0, RB * (D // 16), zero_body, 0)
    z0 = s * ZROWS_PER_S
    pltpu.sync_copy(rows_v.at[pl.ds(0, RB)], acc_sh.at[pl.ds(z0, RB)])
    pltpu.sync_copy(rows_v.at[pl.ds(0, RB)], acc_sh.at[pl.ds(z0 + RB, RB)])
    pltpu.sync_copy(rows_v.at[pl.ds(0, ZROWS_PER_S - 2 * RB)],
                    acc_sh.at[pl.ds(z0 + 2 * RB, ZROWS_PER_S - 2 * RB)])
    plsc.subcore_barrier()

    def super_body(g, _):
        row0 = w * ROWS_PER_W + g * SUPER
        pltpu.sync_copy(src_hbm.at[pl.ds(row0, SUPER)], src_v)
        pltpu.sync_copy(dst_hbm.at[pl.ds(row0, SUPER)], dst_v)
        for grp in range(SUPER // GROUP):
            cps = [
                pltpu.async_copy(h_hbm.at[src_v.at[grp * GROUP + j]],
                                 rows_v.at[pl.ds(j * CHUNK, CHUNK)], sem)
                for j in range(GROUP)
            ]
            for cp in cps:
                cp.wait()
            for j in range(GROUP):
                pltpu.sync_copy(rows_v.at[pl.ds(j * CHUNK, CHUNK)],
                                acc_sh.at[dst_v.at[grp * GROUP + j]], add=True)
        return 0

    lax.fori_loop(0, SUPERS_PER_W, super_body, 0)
    plsc.subcore_barrier()

    @pl.when(s < NS - 1)
    def _():
        r0 = s * WB_ROWS
        pltpu.sync_copy(acc_sh.at[pl.ds(r0, WB_ROWS)],
                        out_hbm.at[c, pl.ds(r0, WB_ROWS)])

    @pl.when(s == NS - 1)
    def _():
        r0 = (NS - 1) * WB_ROWS
        pltpu.sync_copy(acc_sh.at[pl.ds(r0, WB_LAST)],
                        out_hbm.at[c, pl.ds(r0, WB_LAST)])


_agg = pl.kernel(
    _agg_body,
    out_type=jax.ShapeDtypeStruct((NC, N, D), jnp.float32),
    mesh=plsc.VectorSubcoreMesh(core_axis_name="c", subcore_axis_name="s"),
    scratch_types=[
        pltpu.VMEM((SUPER, CHUNK), jnp.int32),
        pltpu.VMEM((SUPER, CHUNK), jnp.int32),
        pltpu.VMEM((RB, D), jnp.float32),
        pltpu.VMEM_SHARED((N_ACC, D), jnp.float32),
        pltpu.SemaphoreType.DMA,
    ],
)


_BN = 1000


def _combine_body(h_ref, p_ref, w_ref, b_ref, o_ref):
    z = h_ref[...] + p_ref[0] + p_ref[1]
    y = jnp.dot(z, w_ref[...], preferred_element_type=jnp.float32) + b_ref[...]
    o_ref[...] = jnp.where(y > 0, y, jnp.exp(y) - 1.0)


def _combine(h, p, wT, b):
    nb = N // _BN
    return pl.pallas_call(
        _combine_body,
        grid=(nb,),
        in_specs=[
            pl.BlockSpec((_BN, D), lambda i: (i, 0)),
            pl.BlockSpec((NC, _BN, D), lambda i: (0, i, 0)),
            pl.BlockSpec((D, D), lambda i: (0, 0)),
            pl.BlockSpec((1, D), lambda i: (0, 0)),
        ],
        out_specs=pl.BlockSpec((_BN, D), lambda i: (i, 0)),
        out_shape=jax.ShapeDtypeStruct((N, D), jnp.float32),
    )(h, p, wT, b)


OUT = 2


def _pool_body(h_ref, b_ref, wo1_ref, bo1_ref, wo2_ref, bo2_ref, o_ref, acc_ref):
    i = pl.program_id(0)

    @pl.when(i == 0)
    def _():
        acc_ref[...] = jnp.zeros_like(acc_ref)

    seg = b_ref[0, 0, :]
    onehot = (lax.broadcasted_iota(jnp.int32, (G, _BN), 0) == seg[None, :])
    acc_ref[...] += jnp.dot(onehot.astype(jnp.float32), h_ref[...],
                            preferred_element_type=jnp.float32)

    @pl.when(i == N // _BN - 1)
    def _():
        t = jnp.dot(acc_ref[...], wo1_ref[...],
                    preferred_element_type=jnp.float32) + bo1_ref[...]
        o_ref[...] = jnp.dot(t, wo2_ref[...],
                             preferred_element_type=jnp.float32) + bo2_ref[...]


def _pool(h, batch3d, wo1T, bo1, wo2T, bo2):
    nb = N // _BN
    return pl.pallas_call(
        _pool_body,
        grid=(nb,),
        in_specs=[
            pl.BlockSpec((_BN, D), lambda i: (i, 0)),
            pl.BlockSpec((1, 1, _BN), lambda i: (i, 0, 0)),
            pl.BlockSpec((D, D), lambda i: (0, 0)),
            pl.BlockSpec((1, D), lambda i: (0, 0)),
            pl.BlockSpec((D, OUT), lambda i: (0, 0)),
            pl.BlockSpec((1, OUT), lambda i: (0, 0)),
        ],
        out_specs=pl.BlockSpec((G, OUT), lambda i: (0, 0)),
        out_shape=jax.ShapeDtypeStruct((G, OUT), jnp.float32),
        scratch_shapes=[pltpu.VMEM((G, D), jnp.float32)],
    )(h, batch3d, wo1T, bo1, wo2T, bo2)


def kernel(x, edge_index, batch, W1, b1, W2, b2, W3, b3, Wo1, bo1, Wo2, bo2):
    src = edge_index[0]
    dst = edge_index[1]
    pad = E_PAD - E
    src2d = jnp.concatenate([src, jnp.zeros((pad,), jnp.int32)]).reshape(ROWS_TOTAL, CHUNK)
    dst2d = jnp.concatenate([dst, jnp.full((pad,), N, jnp.int32)]).reshape(ROWS_TOTAL, CHUNK)
    batch3d = batch.reshape(N // _BN, 1, _BN)

    h = x
    for W, b in ((W1, b1), (W2, b2), (W3, b3)):
        part = _agg(h, src2d, dst2d)
        h = _combine(h, part, W.T, b.reshape(1, D))
    return _pool(h, batch3d, Wo1.T, bo1.reshape(1, D), Wo2.T, bo2.reshape(1, OUT))


# speedup vs baseline: 2.9775x; 1.0960x over previous
import jax
import jax.numpy as jnp
from jax import lax
from jax.experimental import pallas as pl
from jax.experimental.pallas import tpu as pltpu
from jax.experimental.pallas import tpu_sc as plsc

N = 10000
E = 320000
D = 128
G = 64

NC = 2
NS = 16
NW = NC * NS

CHUNK = 128
SUPER = 16
EDGES_PER_SUPER = CHUNK * SUPER * NW
E_PAD = ((E + EDGES_PER_SUPER - 1) // EDGES_PER_SUPER) * EDGES_PER_SUPER
ROWS_TOTAL = E_PAD // CHUNK
ROWS_PER_W = ROWS_TOTAL // NW
SUPERS_PER_W = ROWS_PER_W // SUPER
N_ACC = 10240
ZROWS_PER_S = N_ACC // NS
WB_ROWS = 624
WB_LAST = N - (NS - 1) * WB_ROWS


def _agg_body(h_hbm, src_hbm, dst_hbm, out_hbm, src_v, dst_v, buf_a, buf_b,
              acc_sh, sem_ga, sem_gb):
    c = lax.axis_index("c")
    s = lax.axis_index("s")
    w = c * NS + s

    def zero_body(i, _):
        r = i // (D // 16)
        cc = i % (D // 16)
        buf_a[r, pl.ds(cc * 16, 16)] = jnp.zeros((16,), jnp.float32)
        return 0

    lax.fori_loop(0, CHUNK * (D // 16), zero_body, 0)
    z0 = s * ZROWS_PER_S
    for kk in range(ZROWS_PER_S // CHUNK):
        pltpu.sync_copy(buf_a.at[pl.ds(0, CHUNK)],
                        acc_sh.at[pl.ds(z0 + kk * CHUNK, CHUNK)])
    plsc.subcore_barrier()

    bufs = [(buf_a, sem_ga, None), (buf_b, sem_gb, None)]

    def super_body(g, _):
        row0 = w * ROWS_PER_W + g * SUPER
        pltpu.sync_copy(src_hbm.at[pl.ds(row0, SUPER)], src_v)
        pltpu.sync_copy(dst_hbm.at[pl.ds(row0, SUPER)], dst_v)
        gcp = [None, None]
        gcp[0] = pltpu.async_copy(h_hbm.at[src_v.at[0]], buf_a, sem_ga)
        for j in range(SUPER):
            b = j % 2
            nb = 1 - b
            buf, _, _ = bufs[b]
            nbuf, nsem_g, _ = bufs[nb]
            if j + 1 < SUPER:
                gcp[nb] = pltpu.async_copy(h_hbm.at[src_v.at[j + 1]],
                                           nbuf, nsem_g)
            gcp[b].wait()
            pltpu.sync_copy(buf, acc_sh.at[dst_v.at[j]], add=True)
        return 0

    lax.fori_loop(0, SUPERS_PER_W, super_body, 0)
    plsc.subcore_barrier()

    @pl.when(s < NS - 1)
    def _():
        r0 = s * WB_ROWS
        pltpu.sync_copy(acc_sh.at[pl.ds(r0, WB_ROWS)],
                        out_hbm.at[c, pl.ds(r0, WB_ROWS)])

    @pl.when(s == NS - 1)
    def _():
        r0 = (NS - 1) * WB_ROWS
        pltpu.sync_copy(acc_sh.at[pl.ds(r0, WB_LAST)],
                        out_hbm.at[c, pl.ds(r0, WB_LAST)])


_agg = pl.kernel(
    _agg_body,
    out_type=jax.ShapeDtypeStruct((NC, N, D), jnp.float32),
    mesh=plsc.VectorSubcoreMesh(core_axis_name="c", subcore_axis_name="s"),
    scratch_types=[
        pltpu.VMEM((SUPER, CHUNK), jnp.int32),
        pltpu.VMEM((SUPER, CHUNK), jnp.int32),
        pltpu.VMEM((CHUNK, D), jnp.float32),
        pltpu.VMEM((CHUNK, D), jnp.float32),
        pltpu.VMEM_SHARED((N_ACC, D), jnp.float32),
        pltpu.SemaphoreType.DMA,
        pltpu.SemaphoreType.DMA,
    ],
)


_BN = 1000


def _combine_body(h_ref, p_ref, w_ref, b_ref, o_ref):
    z = h_ref[...] + p_ref[0] + p_ref[1]
    y = jnp.dot(z, w_ref[...], preferred_element_type=jnp.float32) + b_ref[...]
    o_ref[...] = jnp.where(y > 0, y, jnp.exp(y) - 1.0)


def _combine(h, p, wT, b):
    nb = N // _BN
    return pl.pallas_call(
        _combine_body,
        grid=(nb,),
        in_specs=[
            pl.BlockSpec((_BN, D), lambda i: (i, 0)),
            pl.BlockSpec((NC, _BN, D), lambda i: (0, i, 0)),
            pl.BlockSpec((D, D), lambda i: (0, 0)),
            pl.BlockSpec((1, D), lambda i: (0, 0)),
        ],
        out_specs=pl.BlockSpec((_BN, D), lambda i: (i, 0)),
        out_shape=jax.ShapeDtypeStruct((N, D), jnp.float32),
    )(h, p, wT, b)


OUT = 2


def _pool_body(h_ref, b_ref, wo1_ref, bo1_ref, wo2_ref, bo2_ref, o_ref, acc_ref):
    i = pl.program_id(0)

    @pl.when(i == 0)
    def _():
        acc_ref[...] = jnp.zeros_like(acc_ref)

    seg = b_ref[0, 0, :]
    onehot = (lax.broadcasted_iota(jnp.int32, (G, _BN), 0) == seg[None, :])
    acc_ref[...] += jnp.dot(onehot.astype(jnp.float32), h_ref[...],
                            preferred_element_type=jnp.float32)

    @pl.when(i == N // _BN - 1)
    def _():
        t = jnp.dot(acc_ref[...], wo1_ref[...],
                    preferred_element_type=jnp.float32) + bo1_ref[...]
        o_ref[...] = jnp.dot(t, wo2_ref[...],
                             preferred_element_type=jnp.float32) + bo2_ref[...]


def _pool(h, batch3d, wo1T, bo1, wo2T, bo2):
    nb = N // _BN
    return pl.pallas_call(
        _pool_body,
        grid=(nb,),
        in_specs=[
            pl.BlockSpec((_BN, D), lambda i: (i, 0)),
            pl.BlockSpec((1, 1, _BN), lambda i: (i, 0, 0)),
            pl.BlockSpec((D, D), lambda i: (0, 0)),
            pl.BlockSpec((1, D), lambda i: (0, 0)),
            pl.BlockSpec((D, OUT), lambda i: (0, 0)),
            pl.BlockSpec((1, OUT), lambda i: (0, 0)),
        ],
        out_specs=pl.BlockSpec((G, OUT), lambda i: (0, 0)),
        out_shape=jax.ShapeDtypeStruct((G, OUT), jnp.float32),
        scratch_shapes=[pltpu.VMEM((G, D), jnp.float32)],
    )(h, batch3d, wo1T, bo1, wo2T, bo2)


def kernel(x, edge_index, batch, W1, b1, W2, b2, W3, b3, Wo1, bo1, Wo2, bo2):
    src = edge_index[0]
    dst = edge_index[1]
    pad = E_PAD - E
    src2d = jnp.concatenate([src, jnp.zeros((pad,), jnp.int32)]).reshape(ROWS_TOTAL, CHUNK)
    dst2d = jnp.concatenate([dst, jnp.full((pad,), N, jnp.int32)]).reshape(ROWS_TOTAL, CHUNK)
    batch3d = batch.reshape(N // _BN, 1, _BN)

    h = x
    for W, b in ((W1, b1), (W2, b2), (W3, b3)):
        part = _agg(h, src2d, dst2d)
        h = _combine(h, part, W.T, b.reshape(1, D))
    return _pool(h, batch3d, Wo1.T, bo1.reshape(1, D), Wo2.T, bo2.reshape(1, OUT))


# speedup vs baseline: 2.9792x; 1.0006x over previous
import jax
import jax.numpy as jnp
from jax import lax
from jax.experimental import pallas as pl
from jax.experimental.pallas import tpu as pltpu
from jax.experimental.pallas import tpu_sc as plsc

N = 10000
E = 320000
D = 128
G = 64

NC = 2
NS = 16
NW = NC * NS

CHUNK = 128
SUPER = 16
EDGES_PER_SUPER = CHUNK * SUPER * NW
E_PAD = ((E + EDGES_PER_SUPER - 1) // EDGES_PER_SUPER) * EDGES_PER_SUPER
ROWS_TOTAL = E_PAD // CHUNK
ROWS_PER_W = ROWS_TOTAL // NW
SUPERS_PER_W = ROWS_PER_W // SUPER
N_ACC = 10240
ZROWS_PER_S = N_ACC // NS
WB_ROWS = 624
WB_LAST = N - (NS - 1) * WB_ROWS


def _agg_body(h_hbm, src_hbm, dst_hbm, out_hbm, src_v, dst_v, buf_a, buf_b,
              acc_sh, sem_ga, sem_gb):
    c = lax.axis_index("c")
    s = lax.axis_index("s")
    w = c * NS + s

    def zero_body(i, _):
        r = i // (D // 16)
        cc = i % (D // 16)
        buf_a[r, pl.ds(cc * 16, 16)] = jnp.zeros((16,), jnp.float32)
        return 0

    lax.fori_loop(0, CHUNK * (D // 16), zero_body, 0)
    z0 = s * ZROWS_PER_S
    for kk in range(ZROWS_PER_S // CHUNK):
        pltpu.sync_copy(buf_a.at[pl.ds(0, CHUNK)],
                        acc_sh.at[pl.ds(z0 + kk * CHUNK, CHUNK)])
    plsc.subcore_barrier()

    bufs = [(buf_a, sem_ga), (buf_b, sem_gb)]

    def super_body(g, _):
        row0 = w * ROWS_PER_W + g * SUPER
        pltpu.sync_copy(src_hbm.at[pl.ds(row0, SUPER)], src_v)
        pltpu.sync_copy(dst_hbm.at[pl.ds(row0, SUPER)], dst_v)
        gcp = [None, None]
        gcp[0] = pltpu.async_copy(h_hbm.at[src_v.at[0]], buf_a, sem_ga)
        for j in range(SUPER):
            b = j % 2
            nb = 1 - b
            buf, _ = bufs[b]
            nbuf, nsem_g = bufs[nb]
            if j + 1 < SUPER:
                gcp[nb] = pltpu.async_copy(h_hbm.at[src_v.at[j + 1]],
                                           nbuf, nsem_g)
            gcp[b].wait()
            pltpu.sync_copy(buf, acc_sh.at[dst_v.at[j]], add=True)
        return 0

    lax.fori_loop(0, SUPERS_PER_W, super_body, 0)
    plsc.subcore_barrier()

    @pl.when(s < NS - 1)
    def _():
        r0 = s * WB_ROWS
        pltpu.sync_copy(acc_sh.at[pl.ds(r0, WB_ROWS)],
                        out_hbm.at[c, pl.ds(r0, WB_ROWS)])

    @pl.when(s == NS - 1)
    def _():
        r0 = (NS - 1) * WB_ROWS
        pltpu.sync_copy(acc_sh.at[pl.ds(r0, WB_LAST)],
                        out_hbm.at[c, pl.ds(r0, WB_LAST)])


_agg = pl.kernel(
    _agg_body,
    out_type=jax.ShapeDtypeStruct((NC, N, D), jnp.float32),
    mesh=plsc.VectorSubcoreMesh(core_axis_name="c", subcore_axis_name="s"),
    scratch_types=[
        pltpu.VMEM((SUPER, CHUNK), jnp.int32),
        pltpu.VMEM((SUPER, CHUNK), jnp.int32),
        pltpu.VMEM((CHUNK, D), jnp.float32),
        pltpu.VMEM((CHUNK, D), jnp.float32),
        pltpu.VMEM_SHARED((N_ACC, D), jnp.float32),
        pltpu.SemaphoreType.DMA,
        pltpu.SemaphoreType.DMA,
    ],
)


_BN = 1000


def _combine_body(h_ref, p_ref, w_ref, b_ref, o_ref):
    z = h_ref[...] + p_ref[0] + p_ref[1]
    y = jnp.dot(z, w_ref[...], preferred_element_type=jnp.float32) + b_ref[...]
    o_ref[...] = jnp.where(y > 0, y, jnp.exp(y) - 1.0)


def _combine(h, p, wT, b):
    nb = N // _BN
    return pl.pallas_call(
        _combine_body,
        grid=(nb,),
        in_specs=[
            pl.BlockSpec((_BN, D), lambda i: (i, 0)),
            pl.BlockSpec((NC, _BN, D), lambda i: (0, i, 0)),
            pl.BlockSpec((D, D), lambda i: (0, 0)),
            pl.BlockSpec((1, D), lambda i: (0, 0)),
        ],
        out_specs=pl.BlockSpec((_BN, D), lambda i: (i, 0)),
        out_shape=jax.ShapeDtypeStruct((N, D), jnp.float32),
    )(h, p, wT, b)


OUT = 2


def _pool_body(h_ref, b_ref, wo1_ref, bo1_ref, wo2_ref, bo2_ref, o_ref, acc_ref):
    i = pl.program_id(0)

    @pl.when(i == 0)
    def _():
        acc_ref[...] = jnp.zeros_like(acc_ref)

    seg = b_ref[0, 0, :]
    onehot = (lax.broadcasted_iota(jnp.int32, (G, _BN), 0) == seg[None, :])
    acc_ref[...] += jnp.dot(onehot.astype(jnp.float32), h_ref[...],
                            preferred_element_type=jnp.float32)

    @pl.when(i == N // _BN - 1)
    def _():
        t = jnp.dot(acc_ref[...], wo1_ref[...],
                    preferred_element_type=jnp.float32) + bo1_ref[...]
        o_ref[...] = jnp.dot(t, wo2_ref[...],
                             preferred_element_type=jnp.float32) + bo2_ref[...]


def _pool(h, batch3d, wo1T, bo1, wo2T, bo2):
    nb = N // _BN
    return pl.pallas_call(
        _pool_body,
        grid=(nb,),
        in_specs=[
            pl.BlockSpec((_BN, D), lambda i: (i, 0)),
            pl.BlockSpec((1, 1, _BN), lambda i: (i, 0, 0)),
            pl.BlockSpec((D, D), lambda i: (0, 0)),
            pl.BlockSpec((1, D), lambda i: (0, 0)),
            pl.BlockSpec((D, OUT), lambda i: (0, 0)),
            pl.BlockSpec((1, OUT), lambda i: (0, 0)),
        ],
        out_specs=pl.BlockSpec((G, OUT), lambda i: (0, 0)),
        out_shape=jax.ShapeDtypeStruct((G, OUT), jnp.float32),
        scratch_shapes=[pltpu.VMEM((G, D), jnp.float32)],
    )(h, batch3d, wo1T, bo1, wo2T, bo2)


def kernel(x, edge_index, batch, W1, b1, W2, b2, W3, b3, Wo1, bo1, Wo2, bo2):
    src = edge_index[0]
    dst = edge_index[1]
    pad = E_PAD - E
    src2d = jnp.concatenate([src, jnp.zeros((pad,), jnp.int32)]).reshape(ROWS_TOTAL, CHUNK)
    dst2d = jnp.concatenate([dst, jnp.full((pad,), N, jnp.int32)]).reshape(ROWS_TOTAL, CHUNK)
    batch3d = batch.reshape(N // _BN, 1, _BN)

    h = x
    for W, b in ((W1, b1), (W2, b2), (W3, b3)):
        part = _agg(h, src2d, dst2d)
        h = _combine(h, part, W.T, b.reshape(1, D))
    return _pool(h, batch3d, Wo1.T, bo1.reshape(1, D), Wo2.T, bo2.reshape(1, OUT))


# speedup vs baseline: 6.9575x; 2.3354x over previous
import jax
import jax.numpy as jnp
from jax import lax
from jax.experimental import pallas as pl
from jax.experimental.pallas import tpu as pltpu
from jax.experimental.pallas import tpu_sc as plsc

N = 10000
E = 320000
D = 128
G = 64

NC = 2
NS = 16
NW = NC * NS

CHUNK = 128
SUPER = 16
EDGES_PER_SUPER = CHUNK * SUPER * NW
E_PAD = ((E + EDGES_PER_SUPER - 1) // EDGES_PER_SUPER) * EDGES_PER_SUPER
ROWS_TOTAL = E_PAD // CHUNK
ROWS_PER_W = ROWS_TOTAL // NW
SUPERS_PER_W = ROWS_PER_W // SUPER
N_ACC = 10240
ZROWS_PER_S = N_ACC // NS
WB_ROWS = 624
WB_LAST = N - (NS - 1) * WB_ROWS


def _gather_body(h_hbm, src_hbm, em_hbm, src_v, buf_a, buf_b, h_sh,
                 sem_ga, sem_gb, sem_wa, sem_wb):
    c = lax.axis_index("c")
    s = lax.axis_index("s")
    w = c * NS + s

    @pl.when(s < NS - 1)
    def _():
        r0 = s * WB_ROWS
        pltpu.sync_copy(h_hbm.at[pl.ds(r0, WB_ROWS)],
                        h_sh.at[pl.ds(r0, WB_ROWS)])

    @pl.when(s == NS - 1)
    def _():
        r0 = (NS - 1) * WB_ROWS
        pltpu.sync_copy(h_hbm.at[pl.ds(r0, WB_LAST)],
                        h_sh.at[pl.ds(r0, WB_LAST)])

    plsc.subcore_barrier()

    bufs = [(buf_a, sem_ga, sem_wa), (buf_b, sem_gb, sem_wb)]

    def super_body(g, _):
        row0 = w * ROWS_PER_W + g * SUPER
        pltpu.sync_copy(src_hbm.at[pl.ds(row0, SUPER)], src_v)
        gcp = [None, None]
        wcp = [None, None]
        gcp[0] = pltpu.async_copy(h_sh.at[src_v.at[0]], buf_a, sem_ga)
        for j in range(SUPER):
            b = j % 2
            nb = 1 - b
            buf, _, sem_w = bufs[b]
            nbuf, nsem_g, _ = bufs[nb]
            if wcp[nb] is not None:
                wcp[nb].wait()
                wcp[nb] = None
            if j + 1 < SUPER:
                gcp[nb] = pltpu.async_copy(h_sh.at[src_v.at[j + 1]],
                                           nbuf, nsem_g)
            gcp[b].wait()
            wcp[b] = pltpu.async_copy(
                buf, em_hbm.at[pl.ds((row0 + j) * CHUNK, CHUNK)], sem_w)
        for b in (0, 1):
            if wcp[b] is not None:
                wcp[b].wait()
        return 0

    lax.fori_loop(0, SUPERS_PER_W, super_body, 0)


_gatherk = pl.kernel(
    _gather_body,
    out_type=jax.ShapeDtypeStruct((E_PAD, D), jnp.float32),
    mesh=plsc.VectorSubcoreMesh(core_axis_name="c", subcore_axis_name="s"),
    scratch_types=[
        pltpu.VMEM((SUPER, CHUNK), jnp.int32),
        pltpu.VMEM((CHUNK, D), jnp.float32),
        pltpu.VMEM((CHUNK, D), jnp.float32),
        pltpu.VMEM_SHARED((N, D), jnp.float32),
        pltpu.SemaphoreType.DMA,
        pltpu.SemaphoreType.DMA,
        pltpu.SemaphoreType.DMA,
        pltpu.SemaphoreType.DMA,
    ],
)


def _agg_body(em_hbm, dst_hbm, out_hbm, dst_v, buf_a, buf_b,
              acc_sh, sem_ga, sem_gb):
    c = lax.axis_index("c")
    s = lax.axis_index("s")
    w = c * NS + s

    def zero_body(i, _):
        r = i // (D // 16)
        cc = i % (D // 16)
        buf_a[r, pl.ds(cc * 16, 16)] = jnp.zeros((16,), jnp.float32)
        return 0

    lax.fori_loop(0, CHUNK * (D // 16), zero_body, 0)
    z0 = s * ZROWS_PER_S
    for kk in range(ZROWS_PER_S // CHUNK):
        pltpu.sync_copy(buf_a.at[pl.ds(0, CHUNK)],
                        acc_sh.at[pl.ds(z0 + kk * CHUNK, CHUNK)])
    plsc.subcore_barrier()

    bufs = [(buf_a, sem_ga), (buf_b, sem_gb)]

    def super_body(g, _):
        row0 = w * ROWS_PER_W + g * SUPER
        pltpu.sync_copy(dst_hbm.at[pl.ds(row0, SUPER)], dst_v)
        gcp = [None, None]
        gcp[0] = pltpu.async_copy(em_hbm.at[pl.ds(row0 * CHUNK, CHUNK)],
                                  buf_a, sem_ga)
        for j in range(SUPER):
            b = j % 2
            nb = 1 - b
            buf, _ = bufs[b]
            nbuf, nsem_g = bufs[nb]
            if j + 1 < SUPER:
                gcp[nb] = pltpu.async_copy(
                    em_hbm.at[pl.ds((row0 + j + 1) * CHUNK, CHUNK)],
                    nbuf, nsem_g)
            gcp[b].wait()
            pltpu.sync_copy(buf, acc_sh.at[dst_v.at[j]], add=True)
        return 0

    lax.fori_loop(0, SUPERS_PER_W, super_body, 0)
    plsc.subcore_barrier()

    @pl.when(s < NS - 1)
    def _():
        r0 = s * WB_ROWS
        pltpu.sync_copy(acc_sh.at[pl.ds(r0, WB_ROWS)],
                        out_hbm.at[c, pl.ds(r0, WB_ROWS)])

    @pl.when(s == NS - 1)
    def _():
        r0 = (NS - 1) * WB_ROWS
        pltpu.sync_copy(acc_sh.at[pl.ds(r0, WB_LAST)],
                        out_hbm.at[c, pl.ds(r0, WB_LAST)])


_agg = pl.kernel(
    _agg_body,
    out_type=jax.ShapeDtypeStruct((NC, N, D), jnp.float32),
    mesh=plsc.VectorSubcoreMesh(core_axis_name="c", subcore_axis_name="s"),
    scratch_types=[
        pltpu.VMEM((SUPER, CHUNK), jnp.int32),
        pltpu.VMEM((CHUNK, D), jnp.float32),
        pltpu.VMEM((CHUNK, D), jnp.float32),
        pltpu.VMEM_SHARED((N_ACC, D), jnp.float32),
        pltpu.SemaphoreType.DMA,
        pltpu.SemaphoreType.DMA,
    ],
)


_BN = 1000


def _combine_body(h_ref, p_ref, w_ref, b_ref, o_ref):
    z = h_ref[...] + p_ref[0] + p_ref[1]
    y = jnp.dot(z, w_ref[...], preferred_element_type=jnp.float32) + b_ref[...]
    o_ref[...] = jnp.where(y > 0, y, jnp.exp(y) - 1.0)


def _combine(h, p, wT, b):
    nb = N // _BN
    return pl.pallas_call(
        _combine_body,
        grid=(nb,),
        in_specs=[
            pl.BlockSpec((_BN, D), lambda i: (i, 0)),
            pl.BlockSpec((NC, _BN, D), lambda i: (0, i, 0)),
            pl.BlockSpec((D, D), lambda i: (0, 0)),
            pl.BlockSpec((1, D), lambda i: (0, 0)),
        ],
        out_specs=pl.BlockSpec((_BN, D), lambda i: (i, 0)),
        out_shape=jax.ShapeDtypeStruct((N, D), jnp.float32),
    )(h, p, wT, b)


OUT = 2


def _pool_body(h_ref, b_ref, wo1_ref, bo1_ref, wo2_ref, bo2_ref, o_ref, acc_ref):
    i = pl.program_id(0)

    @pl.when(i == 0)
    def _():
        acc_ref[...] = jnp.zeros_like(acc_ref)

    seg = b_ref[0, 0, :]
    onehot = (lax.broadcasted_iota(jnp.int32, (G, _BN), 0) == seg[None, :])
    acc_ref[...] += jnp.dot(onehot.astype(jnp.float32), h_ref[...],
                            preferred_element_type=jnp.float32)

    @pl.when(i == N // _BN - 1)
    def _():
        t = jnp.dot(acc_ref[...], wo1_ref[...],
                    preferred_element_type=jnp.float32) + bo1_ref[...]
        o_ref[...] = jnp.dot(t, wo2_ref[...],
                             preferred_element_type=jnp.float32) + bo2_ref[...]


def _pool(h, batch3d, wo1T, bo1, wo2T, bo2):
    nb = N // _BN
    return pl.pallas_call(
        _pool_body,
        grid=(nb,),
        in_specs=[
            pl.BlockSpec((_BN, D), lambda i: (i, 0)),
            pl.BlockSpec((1, 1, _BN), lambda i: (i, 0, 0)),
            pl.BlockSpec((D, D), lambda i: (0, 0)),
            pl.BlockSpec((1, D), lambda i: (0, 0)),
            pl.BlockSpec((D, OUT), lambda i: (0, 0)),
            pl.BlockSpec((1, OUT), lambda i: (0, 0)),
        ],
        out_specs=pl.BlockSpec((G, OUT), lambda i: (0, 0)),
        out_shape=jax.ShapeDtypeStruct((G, OUT), jnp.float32),
        scratch_shapes=[pltpu.VMEM((G, D), jnp.float32)],
    )(h, batch3d, wo1T, bo1, wo2T, bo2)


def kernel(x, edge_index, batch, W1, b1, W2, b2, W3, b3, Wo1, bo1, Wo2, bo2):
    src = edge_index[0]
    dst = edge_index[1]
    pad = E_PAD - E
    src2d = jnp.concatenate([src, jnp.zeros((pad,), jnp.int32)]).reshape(ROWS_TOTAL, CHUNK)
    dst2d = jnp.concatenate([dst, jnp.full((pad,), N, jnp.int32)]).reshape(ROWS_TOTAL, CHUNK)
    batch3d = batch.reshape(N // _BN, 1, _BN)

    h = x
    for W, b in ((W1, b1), (W2, b2), (W3, b3)):
        em = _gatherk(h, src2d)
        part = _agg(em, dst2d)
        h = _combine(h, part, W.T, b.reshape(1, D))
    return _pool(h, batch3d, Wo1.T, bo1.reshape(1, D), Wo2.T, bo2.reshape(1, OUT))
